# baseline (speedup 1.0000x reference)
import functools

import jax
import jax.numpy as jnp
from jax import lax
from jax.experimental import pallas as pl
from jax.experimental.pallas import tpu as pltpu
from jax.experimental.pallas import tpu_sc as plsc

_N = 1000
_B = 4096
_T = 20
_NC = 2
_NS = 16
_NW = _NC * _NS
_BPW = 256
_NP = _B // _BPW
_JPW = _T // 2
_L = 16
_CSTART = (0, 248, 496, 744)
_CWIDTH = (248, 248, 248, 256)


@functools.partial(
    pl.kernel,
    out_type=jax.ShapeDtypeStruct((_T, _N, _B), jnp.float32),
    mesh=plsc.VectorSubcoreMesh(core_axis_name="c", subcore_axis_name="s"),
    compiler_params=pltpu.CompilerParams(needs_layout_passes=False),
    scratch_types=[
        pltpu.VMEM((248, _BPW), jnp.float32),
        pltpu.VMEM((256, _BPW), jnp.float32),
        pltpu.VMEM((_BPW,), jnp.int32),
        pltpu.VMEM((_BPW,), jnp.int32),
        pltpu.SemaphoreType.DMA,
        pltpu.SemaphoreType.DMA,
    ],
)
def _one_hot_sc(idx_hbm, zeros_hbm, out_hbm, buf0, buf1, idxv0, idxv1,
                sem0, sem1):
    wid = lax.axis_index("s") * _NC + lax.axis_index("c")
    panel = wid // 2
    jpar = wid % 2
    b0 = panel * _BPW

    bufs = (buf0, buf1)
    idxvs = (idxv0, idxv1)
    sems = (sem0, sem1)

    pltpu.sync_copy(zeros_hbm.at[pl.ds(0, 248), :], buf0)
    pltpu.sync_copy(zeros_hbm, buf1)

    lane = lax.broadcasted_iota(jnp.int32, (_L,), 0)
    ones16 = jnp.ones((_L,), jnp.float32)
    zeros16 = jnp.zeros((_L,), jnp.float32)

    def scatter_block(buf, idxv, c0, cw, val):
        for m in range(_BPW // _L):
            bl = lane + m * _L
            cls = idxv[pl.ds(m * _L, _L)]
            mask = (cls >= c0) & (cls < c0 + cw)
            plsc.store_scatter(buf, [cls - c0, bl], val, mask=mask)

    copies = [None, None]
    for k in range(_JPW):
        jj = 2 * k + jpar
        pltpu.sync_copy(idx_hbm.at[pl.ds(jj * _B + b0, _BPW)], idxvs[k % 2])
        for c in range(4):
            s = c % 2
            buf, c0, cw = bufs[s], _CSTART[c], _CWIDTH[c]
            if copies[s] is not None:
                copies[s].wait()
                pc = c - 2 if c >= 2 else c + 2
                pidx = idxvs[k % 2 if c >= 2 else (k - 1) % 2]
                scatter_block(buf, pidx, _CSTART[pc], _CWIDTH[pc], zeros16)
            scatter_block(buf, idxvs[k % 2], c0, cw, ones16)
            copies[s] = pltpu.async_copy(
                buf.at[pl.ds(0, cw), :],
                out_hbm.at[jj, pl.ds(c0, cw), pl.ds(b0, _BPW)],
                sems[s],
            )
    copies[0].wait()
    copies[1].wait()


def kernel(indices, eye):
    del eye
    idx = indices.T.reshape(-1).astype(jnp.int32)
    zeros = jnp.zeros((256, _BPW), jnp.float32)
    out = _one_hot_sc(idx, zeros)
    return jnp.transpose(out, (2, 0, 1))


# speedup vs baseline: 1.0148x; 1.0148x over previous
import functools

import jax
import jax.numpy as jnp
from jax import lax
from jax.experimental import pallas as pl
from jax.experimental.pallas import tpu as pltpu
from jax.experimental.pallas import tpu_sc as plsc

_N = 1000
_B = 4096
_T = 20
_NC = 2
_NS = 16
_NW = _NC * _NS
_BPW = _B // _NW
_L = 16
_C0 = 488
_C1 = _N - _C0


@functools.partial(
    pl.kernel,
    out_type=jax.ShapeDtypeStruct((_T, _N, _B), jnp.float32),
    mesh=plsc.VectorSubcoreMesh(core_axis_name="c", subcore_axis_name="s"),
    compiler_params=pltpu.CompilerParams(needs_layout_passes=False),
    scratch_types=[
        pltpu.VMEM((_C0, _BPW), jnp.float32),
        pltpu.VMEM((_C1, _BPW), jnp.float32),
        pltpu.VMEM((_BPW * _T,), jnp.int32),
        pltpu.SemaphoreType.DMA,
        pltpu.SemaphoreType.DMA,
        pltpu.SemaphoreType.DMA,
    ],
)
def _one_hot_sc(idx_hbm, zeros_hbm, out_hbm, bufa, bufb, idxall,
                sema, semb, semi):
    wid = lax.axis_index("s") * _NC + lax.axis_index("c")
    b0 = wid * _BPW

    init_i = pltpu.async_copy(
        idx_hbm.at[pl.ds(b0 * _T, _BPW * _T)], idxall, semi)
    init_a = pltpu.async_copy(zeros_hbm.at[pl.ds(0, _C0), :], bufa, sema)
    init_b = pltpu.async_copy(zeros_hbm.at[pl.ds(0, _C1), :], bufb, semb)

    lane = lax.broadcasted_iota(jnp.int32, (_L,), 0)
    ones16 = jnp.ones((_L,), jnp.float32)
    zeros16 = jnp.zeros((_L,), jnp.float32)

    def scatter_block(buf, j, c0, cw, val):
        for m in range(_BPW // _L):
            bl = lane + m * _L
            cls = plsc.load_gather(idxall, [bl * _T + j])
            mask = (cls >= c0) & (cls < c0 + cw)
            plsc.store_scatter(buf, [cls - c0, bl], val, mask=mask)

    chunks = ((bufa, 0, _C0, sema), (bufb, _C0, _C1, semb))
    inits = (init_a, init_b)
    copies = [None, None]
    for j in range(_T):
        for s, (buf, c0, cw, sem) in enumerate(chunks):
            if copies[s] is not None:
                copies[s].wait()
                scatter_block(buf, j - 1, c0, cw, zeros16)
            elif s == 0:
                init_i.wait()
                inits[s].wait()
            else:
                inits[s].wait()
            scatter_block(buf, j, c0, cw, ones16)
            copies[s] = pltpu.async_copy(
                buf, out_hbm.at[j, pl.ds(c0, cw), pl.ds(b0, _BPW)], sem
            )
    copies[0].wait()
    copies[1].wait()


def kernel(indices, eye):
    del eye
    idx = indices.reshape(-1).astype(jnp.int32)
    zeros = jnp.zeros((_C1, _BPW), jnp.float32)
    out = _one_hot_sc(idx, zeros)
    return jnp.transpose(out, (2, 0, 1))
